# baseline (speedup 1.0000x reference)
import functools

import jax
import jax.numpy as jnp
from jax.experimental import pallas as pl
from jax.experimental.pallas import tpu as pltpu
from jax.experimental.pallas import tpu_sc as plsc

_B = 2048
_F1 = 25
_F2 = 10
_D = 128
_H = 128

_BB = 64
_R1 = _BB * _F1
_OFF1 = _B
_OFF2 = _B + _B * _F1
_N1 = _B * _F1
_N2 = _B * _F1 * _F2

_BS_SC = -1
_SC_CH = 40


def _f1_avg_matrix():
    rows = jax.lax.broadcasted_iota(jnp.int32, (_BB, _R1), 0)
    cols = jax.lax.broadcasted_iota(jnp.int32, (_BB, _R1), 1)
    return jnp.where(cols // _F1 == rows, 1.0 / _F1, 0.0)


def _sage_block(h0, x1, agg1, ws1, wn1, b1, ws2, wn2, b2):
    dot = functools.partial(jnp.dot, preferred_element_type=jnp.float32)
    s_avg = _f1_avg_matrix()
    new_h1 = jnp.maximum(dot(x1, ws1) + dot(agg1, wn1) + b1, 0.0)
    m1 = dot(s_avg, new_h1)
    agg0 = dot(s_avg, x1)
    new_h0 = jnp.maximum(dot(h0, ws1) + dot(agg0, wn1) + b1, 0.0)
    return dot(new_h0, ws2) + dot(m1, wn2) + b2


def _raw_body(seed_start, nblocks,
              x_ref, h0_ref, ws1_ref, wn1_ref, b1_ref, ws2_ref, wn2_ref,
              b2_ref, out_ref, x1_buf, x2_buf, sem1, sem2, sem3):
    i = pl.program_id(0)
    slot = jax.lax.rem(i, 2)
    nxt = jax.lax.rem(i + 1, 2)

    v1 = x_ref.at[pl.ds(_OFF1, _N1), :]
    v2 = x_ref.at[pl.ds(_OFF2, _N2), :].reshape(_N1, _F2 * _D)
    base = seed_start * _F1
    half = _R1 // 2

    def copies(j, s):
        c1 = pltpu.make_async_copy(
            v1.at[pl.ds(base + j * _R1, _R1), :], x1_buf.at[s], sem1.at[s])
        c2a = pltpu.make_async_copy(
            v2.at[pl.ds(base + j * _R1, half), :],
            x2_buf.at[s, pl.ds(0, half), :], sem2.at[s])
        c2b = pltpu.make_async_copy(
            v2.at[pl.ds(base + j * _R1 + half, half), :],
            x2_buf.at[s, pl.ds(half, half), :], sem3.at[s])
        return c1, c2a, c2b

    @pl.when(i == 0)
    def _prologue():
        for c in copies(0, slot):
            c.start()

    @pl.when(i + 1 < nblocks)
    def _prefetch():
        for c in copies(i + 1, nxt):
            c.start()

    for c in copies(i, slot):
        c.wait()

    x1 = x1_buf[slot]
    x2 = x2_buf[slot]

    agg1 = x2[:, 0:_D]
    for g in range(1, _F2):
        agg1 = agg1 + x2[:, g * _D:(g + 1) * _D]
    agg1 = agg1 * (1.0 / _F2)

    out_ref[...] = _sage_block(
        h0_ref[...], x1, agg1, ws1_ref[...], wn1_ref[...], b1_ref[...],
        ws2_ref[...], wn2_ref[...], b2_ref[...])


def _pre_body(nblocks,
              x_ref, agg1_ref, h0_ref, ws1_ref, wn1_ref, b1_ref, ws2_ref,
              wn2_ref, b2_ref, out_ref, x1_buf, sem1):
    i = pl.program_id(0)
    slot = jax.lax.rem(i, 2)
    nxt = jax.lax.rem(i + 1, 2)

    v1 = x_ref.at[pl.ds(_OFF1, _N1), :]

    def copies(j, s):
        return (pltpu.make_async_copy(
            v1.at[pl.ds(j * _R1, _R1), :], x1_buf.at[s], sem1.at[s]),)

    @pl.when(i == 0)
    def _prologue():
        for c in copies(0, slot):
            c.start()

    @pl.when(i + 1 < nblocks)
    def _prefetch():
        for c in copies(i + 1, nxt):
            c.start()

    for c in copies(i, slot):
        c.wait()

    out_ref[...] = _sage_block(
        h0_ref[...], x1_buf[slot], agg1_ref[...], ws1_ref[...], wn1_ref[...],
        b1_ref[...], ws2_ref[...], wn2_ref[...], b2_ref[...])


_WSPECS = [
    pl.BlockSpec((_D, _H), lambda i: (0, 0)),
    pl.BlockSpec((_D, _H), lambda i: (0, 0)),
    pl.BlockSpec((1, _H), lambda i: (0, 0)),
    pl.BlockSpec((_H, _H), lambda i: (0, 0)),
    pl.BlockSpec((_H, _H), lambda i: (0, 0)),
    pl.BlockSpec((1, _H), lambda i: (0, 0)),
]


def _tc_raw(x, weights, seed_start, nseeds):
    nblocks = nseeds // _BB
    sb = seed_start // _BB
    return pl.pallas_call(
        functools.partial(_raw_body, seed_start, nblocks),
        grid=(nblocks,),
        in_specs=[
            pl.BlockSpec(memory_space=pl.ANY),
            pl.BlockSpec((_BB, _D), lambda i: (sb + i, 0)),
        ] + _WSPECS,
        out_specs=pl.BlockSpec((_BB, _H), lambda i: (i, 0)),
        out_shape=jax.ShapeDtypeStruct((nseeds, _H), jnp.float32),
        scratch_shapes=[
            pltpu.VMEM((2, _R1, _D), jnp.float32),
            pltpu.VMEM((2, _R1, _F2 * _D), jnp.float32),
            pltpu.SemaphoreType.DMA((2,)),
            pltpu.SemaphoreType.DMA((2,)),
            pltpu.SemaphoreType.DMA((2,)),
        ],
        compiler_params=pltpu.CompilerParams(
            dimension_semantics=("arbitrary",),
        ),
    )(x, x, *weights)


def _tc_pre(x, agg1, weights, nseeds):
    nblocks = nseeds // _BB
    return pl.pallas_call(
        functools.partial(_pre_body, nblocks),
        grid=(nblocks,),
        in_specs=[
            pl.BlockSpec(memory_space=pl.ANY),
            pl.BlockSpec((_R1, _D), lambda i: (i, 0)),
            pl.BlockSpec((_BB, _D), lambda i: (i, 0)),
        ] + _WSPECS,
        out_specs=pl.BlockSpec((_BB, _H), lambda i: (i, 0)),
        out_shape=jax.ShapeDtypeStruct((nseeds, _H), jnp.float32),
        scratch_shapes=[
            pltpu.VMEM((2, _R1, _D), jnp.float32),
            pltpu.SemaphoreType.DMA((2,)),
        ],
        compiler_params=pltpu.CompilerParams(
            dimension_semantics=("arbitrary",),
        ),
    )(x, agg1, x, *weights)


def _sc_agg1(x, nseeds):
    nrows = nseeds * _F1
    per_sub = nrows // (_SC_CH * 32)
    mesh = plsc.VectorSubcoreMesh(core_axis_name="c", subcore_axis_name="s")
    cin = _SC_CH * _F2

    @functools.partial(
        pl.kernel,
        out_type=jax.ShapeDtypeStruct((nrows, _D), jnp.float32),
        mesh=mesh,
        scratch_types=[
            pltpu.VMEM((2, _SC_CH * _F2, _D), jnp.float32),
            pltpu.VMEM((2, _SC_CH, _D), jnp.float32),
            pltpu.SemaphoreType.DMA((2,)),
            pltpu.SemaphoreType.DMA((2,)),
        ])
    def sc(x_hbm, o_hbm, in_buf, out_buf, sem_in, sem_out):
        c = jax.lax.axis_index("c")
        s = jax.lax.axis_index("s")
        w = c * 16 + s
        v2 = x_hbm.at[pl.ds(_OFF2, nrows * _F2), :]

        def in_copy(t, slot):
            blk = w * per_sub + t
            return pltpu.make_async_copy(
                v2.at[pl.ds(blk * cin, cin), :], in_buf.at[slot],
                sem_in.at[slot])

        def out_copy(t, slot):
            blk = w * per_sub + t
            return pltpu.make_async_copy(
                out_buf.at[slot], o_hbm.at[pl.ds(blk * _SC_CH, _SC_CH), :],
                sem_out.at[slot])

        in_copy(0, 0).start()

        @pl.loop(0, per_sub)
        def _(t):
            slot = jax.lax.rem(t, 2)
            nslot = jax.lax.rem(t + 1, 2)

            @pl.when(t + 1 < per_sub)
            def _():
                in_copy(t + 1, nslot).start()

            @pl.when(t >= 2)
            def _():
                out_copy(t - 2, slot).wait()

            in_copy(t, slot).wait()

            @pl.loop(0, _SC_CH, step=4)
            def _(i):
                for k in range(4):
                    base = (i + k) * _F2
                    for d in range(_D // 16):
                        sl = pl.ds(16 * d, 16)
                        acc = in_buf[slot, base, sl]
                        for g in range(1, _F2):
                            acc = acc + in_buf[slot, base + g, sl]
                        out_buf[slot, i + k, sl] = acc * (1.0 / _F2)

            out_copy(t, slot).start()

        for j in (per_sub - 2, per_sub - 1):
            if j >= 0:
                out_copy(j, j % 2).wait()

    return sc(x)


@jax.jit
def kernel(inputs, W_self1, W_neigh1, b1, W_self2, W_neigh2, b2):
    weights = (W_self1, W_neigh1, b1.reshape(1, _H),
               W_self2, W_neigh2, b2.reshape(1, _H))
    if _BS_SC == 0:
        return _tc_raw(inputs, weights, 0, _B)
    if _BS_SC == -1:
        agg1_sc = _sc_agg1(inputs, 512)
        out = _tc_raw(inputs, weights, 0, _B)
        return out + 0.0 * agg1_sc[:_B]
    agg1_sc = _sc_agg1(inputs, _BS_SC)
    parts = []
    if _BS_SC < _B:
        out_hi = _tc_raw(inputs, weights, _BS_SC, _B - _BS_SC)
        parts.append(out_hi)
    out_lo = _tc_pre(inputs, agg1_sc, weights, _BS_SC)
    parts.insert(0, out_lo)
    if len(parts) == 1:
        return parts[0]
    return jnp.concatenate(parts, axis=0)


# speedup vs baseline: 1.4561x; 1.4561x over previous
import functools

import jax
import jax.numpy as jnp
from jax.experimental import pallas as pl
from jax.experimental.pallas import tpu as pltpu
from jax.experimental.pallas import tpu_sc as plsc

_B = 2048
_F1 = 25
_F2 = 10
_D = 128
_H = 128

_BB = 64
_R1 = _BB * _F1
_OFF1 = _B
_OFF2 = _B + _B * _F1
_N1 = _B * _F1
_N2 = _B * _F1 * _F2

_BS_SC = 0
_SC_CH = 40


def _f1_avg_matrix():
    rows = jax.lax.broadcasted_iota(jnp.int32, (_BB, _R1), 0)
    cols = jax.lax.broadcasted_iota(jnp.int32, (_BB, _R1), 1)
    return jnp.where(cols // _F1 == rows, 1.0 / _F1, 0.0)


def _sage_block(h0, x1, agg1, ws1, wn1, b1, ws2, wn2, b2):
    dot = functools.partial(jnp.dot, preferred_element_type=jnp.float32)
    s_avg = _f1_avg_matrix()
    new_h1 = jnp.maximum(dot(x1, ws1) + dot(agg1, wn1) + b1, 0.0)
    m1 = dot(s_avg, new_h1)
    agg0 = dot(s_avg, x1)
    new_h0 = jnp.maximum(dot(h0, ws1) + dot(agg0, wn1) + b1, 0.0)
    return dot(new_h0, ws2) + dot(m1, wn2) + b2


def _raw_body(seed_start, nblocks,
              x_ref, h0_ref, ws1_ref, wn1_ref, b1_ref, ws2_ref, wn2_ref,
              b2_ref, out_ref, x1_buf, x2_buf, sem1, sem2, sem3):
    i = pl.program_id(0)
    slot = jax.lax.rem(i, 3)
    nxt = jax.lax.rem(i + 2, 3)

    v1 = x_ref.at[pl.ds(_OFF1, _N1), :]
    v2 = x_ref.at[pl.ds(_OFF2, _N2), :].reshape(_N1, _F2 * _D)
    base = seed_start * _F1
    half = _R1 // 2

    def copies(j, s):
        c1 = pltpu.make_async_copy(
            v1.at[pl.ds(base + j * _R1, _R1), :], x1_buf.at[s], sem1.at[s])
        c2a = pltpu.make_async_copy(
            v2.at[pl.ds(base + j * _R1, half), :],
            x2_buf.at[s, pl.ds(0, half), :], sem2.at[s])
        c2b = pltpu.make_async_copy(
            v2.at[pl.ds(base + j * _R1 + half, half), :],
            x2_buf.at[s, pl.ds(half, half), :], sem3.at[s])
        return c1, c2a, c2b

    @pl.when(i == 0)
    def _prologue():
        for c in copies(0, 0):
            c.start()
        if nblocks > 1:
            for c in copies(1, 1):
                c.start()

    @pl.when(jnp.logical_and(i + 2 < nblocks, i > 0))
    def _prefetch():
        for c in copies(i + 2, nxt):
            c.start()

    @pl.when(jnp.logical_and(i == 0, nblocks > 2))
    def _prefetch0():
        for c in copies(2, 2):
            c.start()

    for c in copies(i, slot):
        c.wait()

    x1 = x1_buf[slot]
    x2 = x2_buf[slot]

    agg1 = x2[:, 0:_D]
    for g in range(1, _F2):
        agg1 = agg1 + x2[:, g * _D:(g + 1) * _D]
    agg1 = agg1 * (1.0 / _F2)

    out_ref[...] = _sage_block(
        h0_ref[...], x1, agg1, ws1_ref[...], wn1_ref[...], b1_ref[...],
        ws2_ref[...], wn2_ref[...], b2_ref[...])


def _pre_body(nblocks,
              x_ref, agg1_ref, h0_ref, ws1_ref, wn1_ref, b1_ref, ws2_ref,
              wn2_ref, b2_ref, out_ref, x1_buf, sem1):
    i = pl.program_id(0)
    slot = jax.lax.rem(i, 2)
    nxt = jax.lax.rem(i + 1, 2)

    v1 = x_ref.at[pl.ds(_OFF1, _N1), :]

    def copies(j, s):
        return (pltpu.make_async_copy(
            v1.at[pl.ds(j * _R1, _R1), :], x1_buf.at[s], sem1.at[s]),)

    @pl.when(i == 0)
    def _prologue():
        for c in copies(0, slot):
            c.start()

    @pl.when(i + 1 < nblocks)
    def _prefetch():
        for c in copies(i + 1, nxt):
            c.start()

    for c in copies(i, slot):
        c.wait()

    out_ref[...] = _sage_block(
        h0_ref[...], x1_buf[slot], agg1_ref[...], ws1_ref[...], wn1_ref[...],
        b1_ref[...], ws2_ref[...], wn2_ref[...], b2_ref[...])


_WSPECS = [
    pl.BlockSpec((_D, _H), lambda i: (0, 0)),
    pl.BlockSpec((_D, _H), lambda i: (0, 0)),
    pl.BlockSpec((1, _H), lambda i: (0, 0)),
    pl.BlockSpec((_H, _H), lambda i: (0, 0)),
    pl.BlockSpec((_H, _H), lambda i: (0, 0)),
    pl.BlockSpec((1, _H), lambda i: (0, 0)),
]


def _tc_raw(x, weights, seed_start, nseeds):
    nblocks = nseeds // _BB
    sb = seed_start // _BB
    return pl.pallas_call(
        functools.partial(_raw_body, seed_start, nblocks),
        grid=(nblocks,),
        in_specs=[
            pl.BlockSpec(memory_space=pl.ANY),
            pl.BlockSpec((_BB, _D), lambda i: (sb + i, 0)),
        ] + _WSPECS,
        out_specs=pl.BlockSpec((_BB, _H), lambda i: (i, 0)),
        out_shape=jax.ShapeDtypeStruct((nseeds, _H), jnp.float32),
        scratch_shapes=[
            pltpu.VMEM((3, _R1, _D), jnp.float32),
            pltpu.VMEM((3, _R1, _F2 * _D), jnp.float32),
            pltpu.SemaphoreType.DMA((3,)),
            pltpu.SemaphoreType.DMA((3,)),
            pltpu.SemaphoreType.DMA((3,)),
        ],
        compiler_params=pltpu.CompilerParams(
            dimension_semantics=("arbitrary",),
        ),
    )(x, x, *weights)


def _tc_pre(x, agg1, weights, nseeds):
    nblocks = nseeds // _BB
    return pl.pallas_call(
        functools.partial(_pre_body, nblocks),
        grid=(nblocks,),
        in_specs=[
            pl.BlockSpec(memory_space=pl.ANY),
            pl.BlockSpec((_R1, _D), lambda i: (i, 0)),
            pl.BlockSpec((_BB, _D), lambda i: (i, 0)),
        ] + _WSPECS,
        out_specs=pl.BlockSpec((_BB, _H), lambda i: (i, 0)),
        out_shape=jax.ShapeDtypeStruct((nseeds, _H), jnp.float32),
        scratch_shapes=[
            pltpu.VMEM((2, _R1, _D), jnp.float32),
            pltpu.SemaphoreType.DMA((2,)),
        ],
        compiler_params=pltpu.CompilerParams(
            dimension_semantics=("arbitrary",),
        ),
    )(x, agg1, x, *weights)


def _sc_agg1(x, nseeds):
    nrows = nseeds * _F1
    per_sub = nrows // (_SC_CH * 32)
    mesh = plsc.VectorSubcoreMesh(core_axis_name="c", subcore_axis_name="s")
    cin = _SC_CH * _F2

    @functools.partial(
        pl.kernel,
        out_type=jax.ShapeDtypeStruct((nrows, _D), jnp.float32),
        mesh=mesh,
        scratch_types=[
            pltpu.VMEM((2, _SC_CH * _F2, _D), jnp.float32),
            pltpu.VMEM((2, _SC_CH, _D), jnp.float32),
            pltpu.SemaphoreType.DMA((2,)),
            pltpu.SemaphoreType.DMA((2,)),
        ])
    def sc(x_hbm, o_hbm, in_buf, out_buf, sem_in, sem_out):
        c = jax.lax.axis_index("c")
        s = jax.lax.axis_index("s")
        w = c * 16 + s
        v2 = x_hbm.at[pl.ds(_OFF2, nrows * _F2), :]

        def in_copy(t, slot):
            blk = w * per_sub + t
            return pltpu.make_async_copy(
                v2.at[pl.ds(blk * cin, cin), :], in_buf.at[slot],
                sem_in.at[slot])

        def out_copy(t, slot):
            blk = w * per_sub + t
            return pltpu.make_async_copy(
                out_buf.at[slot], o_hbm.at[pl.ds(blk * _SC_CH, _SC_CH), :],
                sem_out.at[slot])

        in_copy(0, 0).start()

        @pl.loop(0, per_sub)
        def _(t):
            slot = jax.lax.rem(t, 2)
            nslot = jax.lax.rem(t + 1, 2)

            @pl.when(t + 1 < per_sub)
            def _():
                in_copy(t + 1, nslot).start()

            @pl.when(t >= 2)
            def _():
                out_copy(t - 2, slot).wait()

            in_copy(t, slot).wait()

            @pl.loop(0, _SC_CH, step=4)
            def _(i):
                for k in range(4):
                    base = (i + k) * _F2
                    for d in range(_D // 16):
                        sl = pl.ds(16 * d, 16)
                        acc = in_buf[slot, base, sl]
                        for g in range(1, _F2):
                            acc = acc + in_buf[slot, base + g, sl]
                        out_buf[slot, i + k, sl] = acc * (1.0 / _F2)

            out_copy(t, slot).start()

        for j in (per_sub - 2, per_sub - 1):
            if j >= 0:
                out_copy(j, j % 2).wait()

    return sc(x)


@jax.jit
def kernel(inputs, W_self1, W_neigh1, b1, W_self2, W_neigh2, b2):
    weights = (W_self1, W_neigh1, b1.reshape(1, _H),
               W_self2, W_neigh2, b2.reshape(1, _H))
    if _BS_SC == 0:
        return _tc_raw(inputs, weights, 0, _B)
    if _BS_SC == -1:
        agg1_sc = _sc_agg1(inputs, 512)
        out = _tc_raw(inputs, weights, 0, _B)
        return out + 0.0 * agg1_sc[:_B]
    agg1_sc = _sc_agg1(inputs, _BS_SC)
    parts = []
    if _BS_SC < _B:
        out_hi = _tc_raw(inputs, weights, _BS_SC, _B - _BS_SC)
        parts.append(out_hi)
    out_lo = _tc_pre(inputs, agg1_sc, weights, _BS_SC)
    parts.insert(0, out_lo)
    if len(parts) == 1:
        return parts[0]
    return jnp.concatenate(parts, axis=0)


# speedup vs baseline: 1.4723x; 1.0111x over previous
import functools

import jax
import jax.numpy as jnp
from jax.experimental import pallas as pl
from jax.experimental.pallas import tpu as pltpu

_B = 2048
_F1 = 25
_F2 = 10
_D = 128
_H = 128

_BB = 64
_NB = _B // _BB
_R1 = _BB * _F1
_OFF1 = _B
_OFF2 = _B + _B * _F1
_N1 = _B * _F1
_N2 = _B * _F1 * _F2


def _body(x_ref, h0_ref, ws1_ref, wn1_ref, b1_ref, ws2_ref, wn2_ref, b2_ref,
          out_ref, x1_buf, x2_buf, sem1, sem2, sem3):
    i = pl.program_id(0)
    slot = jax.lax.rem(i, 2)
    nxt = jax.lax.rem(i + 1, 2)

    v1 = x_ref.at[pl.ds(_OFF1, _N1), :]
    v2 = x_ref.at[pl.ds(_OFF2, _N2), :].reshape(_N1, _F2 * _D)
    half = _R1 // 2

    def copies(j, s):
        c1 = pltpu.make_async_copy(
            v1.at[pl.ds(j * _R1, _R1), :], x1_buf.at[s], sem1.at[s])
        c2a = pltpu.make_async_copy(
            v2.at[pl.ds(j * _R1, half), :],
            x2_buf.at[s, pl.ds(0, half), :], sem2.at[s])
        c2b = pltpu.make_async_copy(
            v2.at[pl.ds(j * _R1 + half, half), :],
            x2_buf.at[s, pl.ds(half, half), :], sem3.at[s])
        return c1, c2a, c2b

    @pl.when(i == 0)
    def _prologue():
        for c in copies(0, slot):
            c.start()

    @pl.when(i + 1 < _NB)
    def _prefetch():
        for c in copies(i + 1, nxt):
            c.start()

    for c in copies(i, slot):
        c.wait()

    x1 = x1_buf[slot]
    x2 = x2_buf[slot]

    agg1 = x2[:, 0:_D]
    for g in range(1, _F2):
        agg1 = agg1 + x2[:, g * _D:(g + 1) * _D]
    agg1 = agg1 * (1.0 / _F2)

    rows = jax.lax.broadcasted_iota(jnp.int32, (_BB, _R1), 0)
    cols = jax.lax.broadcasted_iota(jnp.int32, (_BB, _R1), 1)
    s_avg = jnp.where(cols // _F1 == rows, 1.0 / _F1, 0.0)

    dot = functools.partial(jnp.dot, preferred_element_type=jnp.float32)
    ws1 = ws1_ref[...]
    wn1 = wn1_ref[...]
    b1 = b1_ref[...]

    new_h1 = jnp.maximum(dot(x1, ws1) + dot(agg1, wn1) + b1, 0.0)
    m1 = dot(s_avg, new_h1)
    agg0 = dot(s_avg, x1)
    h0 = h0_ref[...]
    new_h0 = jnp.maximum(dot(h0, ws1) + dot(agg0, wn1) + b1, 0.0)
    out_ref[...] = (dot(new_h0, ws2_ref[...]) + dot(m1, wn2_ref[...])
                    + b2_ref[...])


@jax.jit
def kernel(inputs, W_self1, W_neigh1, b1, W_self2, W_neigh2, b2):
    out = pl.pallas_call(
        _body,
        grid=(_NB,),
        in_specs=[
            pl.BlockSpec(memory_space=pl.ANY),
            pl.BlockSpec((_BB, _D), lambda i: (i, 0)),
            pl.BlockSpec((_D, _H), lambda i: (0, 0)),
            pl.BlockSpec((_D, _H), lambda i: (0, 0)),
            pl.BlockSpec((1, _H), lambda i: (0, 0)),
            pl.BlockSpec((_H, _H), lambda i: (0, 0)),
            pl.BlockSpec((_H, _H), lambda i: (0, 0)),
            pl.BlockSpec((1, _H), lambda i: (0, 0)),
        ],
        out_specs=pl.BlockSpec((_BB, _H), lambda i: (i, 0)),
        out_shape=jax.ShapeDtypeStruct((_B, _H), jnp.float32),
        scratch_shapes=[
            pltpu.VMEM((2, _R1, _D), jnp.float32),
            pltpu.VMEM((2, _R1, _F2 * _D), jnp.float32),
            pltpu.SemaphoreType.DMA((2,)),
            pltpu.SemaphoreType.DMA((2,)),
            pltpu.SemaphoreType.DMA((2,)),
        ],
        compiler_params=pltpu.CompilerParams(
            dimension_semantics=("arbitrary",),
        ),
    )(inputs, inputs, W_self1, W_neigh1, b1.reshape(1, _H),
      W_self2, W_neigh2, b2.reshape(1, _H))
    return out
